# baseline (speedup 1.0000x reference)
import functools

import jax
import jax.numpy as jnp
from jax import lax
from jax.experimental import pallas as pl
from jax.experimental.pallas import tpu as pltpu
from jax.experimental.pallas import tpu_sc as plsc

NC = 2
NS = 16
NW = NC * NS
CHUNK = 120
NBUF = 3
F0_NUM, F0_DEN = 603, 1000

ROW_BLK = 2000


def _make_segsum(n_rows, h, k0, k1, acc_rows):
    mesh = plsc.VectorSubcoreMesh(core_axis_name="c", subcore_axis_name="s")
    rpt = acc_rows // NS

    @functools.partial(
        pl.kernel,
        mesh=mesh,
        out_type=jax.ShapeDtypeStruct((NC, acc_rows, h), jnp.float32),
        scratch_types=[
            pltpu.VMEM((8, 128), jnp.int32),
            pltpu.VMEM((8, 128), jnp.int32),
            pltpu.VMEM((NBUF, CHUNK, h), jnp.float32),
            pltpu.VMEM_SHARED((acc_rows, h), jnp.float32),
        ] + [pltpu.SemaphoreType.DMA] * (4 * NBUF),
    )
    def segsum(table_hbm, src_hbm, dst_hbm, zeros_hbm, out_hbm,
               sring, dring, rows_v, acc, *sems):
        c = lax.axis_index("c")
        s = lax.axis_index("s")
        wid = s * NC + c
        kc = jnp.where(c == 0, k0, k1)
        gsem = sems[:NBUF]
        ssem = sems[NBUF:2 * NBUF]
        dsem = sems[2 * NBUF:3 * NBUF]
        isem = sems[3 * NBUF:]
        pltpu.sync_copy(zeros_hbm.at[pl.ds(s * rpt, rpt)],
                        acc.at[pl.ds(s * rpt, rpt)])
        plsc.subcore_barrier()

        def sidx_cp(jj, b):
            return pltpu.make_async_copy(src_hbm.at[wid, jj], sring.at[b],
                                         isem[b])

        def didx_cp(jj, b):
            return pltpu.make_async_copy(dst_hbm.at[wid, jj], dring.at[b],
                                         dsem[b])

        def gather_cp(jj, b):
            return pltpu.make_async_copy(
                table_hbm.at[sring.at[b, pl.ds(0, CHUNK)]], rows_v.at[b],
                gsem[b])

        def scatter_cp(b):
            return pltpu.make_async_copy(
                rows_v.at[b], acc.at[dring.at[b, pl.ds(0, CHUNK)]], ssem[b])

        sidx_cp(0, 0).start()
        sidx_cp(1, 1).start()
        sidx_cp(2, 2).start()
        didx_cp(0, 0).start()
        didx_cp(1, 1).start()
        sidx_cp(0, 0).wait()
        gather_cp(0, 0).start()
        sidx_cp(1, 1).wait()
        gather_cp(1, 1).start()

        def body(i, carry):
            j = i * NBUF
            for b in range(NBUF):
                jj = j + b
                bn = (b + 2) % NBUF

                @pl.when(jj >= 1)
                def _():
                    scatter_cp(bn).wait()

                gather_cp(jj, b).wait()
                didx_cp(jj, b).wait()
                pltpu.async_copy(
                    rows_v.at[b], acc.at[dring.at[b, pl.ds(0, CHUNK)]],
                    ssem[b], add=True)

                @pl.when(jj + 3 < kc)
                def _():
                    sidx_cp(jj + 3, b).start()

                @pl.when(jj + 2 < kc)
                def _():
                    didx_cp(jj + 2, bn).start()
                    sidx_cp(jj + 2, bn).wait()
                    gather_cp(jj + 2, bn).start()
            return carry

        lax.fori_loop(0, kc // NBUF, body, 0)
        scatter_cp(2).wait()
        plsc.subcore_barrier()
        pltpu.sync_copy(acc.at[pl.ds(s * rpt, rpt)],
                        out_hbm.at[c, pl.ds(s * rpt, rpt)])

    return segsum


def _mlp1_body(x_ref, a0_ref, a1_ref, w_ref, b_ref, t_ref, stats_ref):
    hcols = t_ref.shape[1]
    hid = x_ref[...] + a0_ref[...] + a1_ref[...]
    t = jnp.dot(hid, w_ref[...], preferred_element_type=jnp.float32) + b_ref[...]
    t_ref[...] = t

    @pl.when(pl.program_id(0) == 0)
    def _():
        stats_ref[...] = jnp.zeros_like(stats_ref)

    sums = jnp.concatenate(
        [jnp.sum(t, axis=0, keepdims=True),
         jnp.sum(t * t, axis=0, keepdims=True),
         jnp.zeros((6, hcols), jnp.float32)],
        axis=0,
    )
    stats_ref[...] += sums


def _mlp1(x, a0, a1, w1, b1, n_rows):
    h = x.shape[1]
    h2 = w1.shape[1]
    grid = n_rows // ROW_BLK
    return pl.pallas_call(
        _mlp1_body,
        grid=(grid,),
        in_specs=[
            pl.BlockSpec((ROW_BLK, h), lambda i: (i, 0)),
            pl.BlockSpec((ROW_BLK, h), lambda i: (i, 0)),
            pl.BlockSpec((ROW_BLK, h), lambda i: (i, 0)),
            pl.BlockSpec((h, h2), lambda i: (0, 0)),
            pl.BlockSpec((1, h2), lambda i: (0, 0)),
        ],
        out_specs=[
            pl.BlockSpec((ROW_BLK, h2), lambda i: (i, 0)),
            pl.BlockSpec((8, h2), lambda i: (0, 0)),
        ],
        out_shape=[
            jax.ShapeDtypeStruct((n_rows, h2), jnp.float32),
            jax.ShapeDtypeStruct((8, h2), jnp.float32),
        ],
    )(x, a0, a1, w1, b1.reshape(1, h2))


def _mlp2_body(t_ref, stats_ref, g_ref, be_ref, w_ref, b_ref, o_ref,
               *, relu, n_rows, final):
    inv_n = 1.0 / n_rows
    m = stats_ref[0:1, :] * inv_n
    v = stats_ref[1:2, :] * inv_n - m * m
    scale = lax.rsqrt(v + 1e-5) * g_ref[...]
    hid = (t_ref[...] - m) * scale + be_ref[...]
    if relu:
        hid = jnp.maximum(hid, 0.0)
    o = jnp.dot(hid, w_ref[...], preferred_element_type=jnp.float32) + b_ref[...]
    o_ref[...] = o


def _mlp2_final_body(t_ref, stats_ref, g_ref, be_ref, w_ref, b_ref,
                     mean_ref, noise_ref, o_ref, *, n_rows):
    inv_n = 1.0 / n_rows
    m = stats_ref[0:1, :] * inv_n
    v = stats_ref[1:2, :] * inv_n - m * m
    scale = lax.rsqrt(v + 1e-5) * g_ref[...]
    hid = (t_ref[...] - m) * scale + be_ref[...]
    o = jnp.dot(hid, w_ref[...], preferred_element_type=jnp.float32) + b_ref[...]
    o_ref[...] = noise_ref[...] * jnp.exp(o) + mean_ref[...]


def _mlp2(t, stats, g, be, w2, b2, relu, n_rows, mean=None, noise=None):
    h2 = t.shape[1]
    h = w2.shape[1]
    grid = n_rows // ROW_BLK
    in_specs = [
        pl.BlockSpec((ROW_BLK, h2), lambda i: (i, 0)),
        pl.BlockSpec((8, h2), lambda i: (0, 0)),
        pl.BlockSpec((1, h2), lambda i: (0, 0)),
        pl.BlockSpec((1, h2), lambda i: (0, 0)),
        pl.BlockSpec((h2, h), lambda i: (0, 0)),
        pl.BlockSpec((1, h), lambda i: (0, 0)),
    ]
    args = [t, stats, g.reshape(1, h2), be.reshape(1, h2), w2, b2.reshape(1, h)]
    if mean is None:
        body = functools.partial(_mlp2_body, relu=relu, n_rows=n_rows, final=False)
    else:
        body = functools.partial(_mlp2_final_body, n_rows=n_rows)
        in_specs += [
            pl.BlockSpec((ROW_BLK, h), lambda i: (i, 0)),
            pl.BlockSpec((ROW_BLK, h), lambda i: (i, 0)),
        ]
        args += [mean, noise]
    return pl.pallas_call(
        body,
        grid=(grid,),
        in_specs=in_specs,
        out_specs=pl.BlockSpec((ROW_BLK, h), lambda i: (i, 0)),
        out_shape=jax.ShapeDtypeStruct((n_rows, h), jnp.float32),
    )(*args)


def kernel(x, edge_index, gaussian_noise, params):
    n, h = x.shape
    e = edge_index.shape[1]
    acc_rows = ((n + NS * 8) // (NS * 8)) * (NS * 8)

    epp = -(-e // NS)
    e0 = (epp * F0_NUM) // F0_DEN
    k0 = -(-(-(-e0 // CHUNK)) // NBUF) * NBUF
    e1 = epp - e0
    k1 = -(-(-(-e1 // CHUNK)) // NBUF) * NBUF

    def slab(vals, fill, e_pt, k):
        v = vals.reshape(NS, e_pt)
        v = jnp.pad(v, ((0, 0), (0, k * CHUNK - e_pt)), constant_values=fill)
        v = v.reshape(NS, k, CHUNK)
        v = jnp.pad(v, ((0, 0), (0, k0 - k), (0, 128 - CHUNK)),
                    constant_values=fill)
        return v

    def split(vals, fill):
        a = slab(vals[:NS * e0], fill, e0, k0)
        b = slab(vals[NS * e0:NS * epp], fill, e1, k1)
        return jnp.stack([a, b], axis=1).reshape(NW, k0, 128)

    ep = NS * epp
    src = split(jnp.concatenate(
        [edge_index[0], jnp.zeros((ep - e,), jnp.int32)]), 0)
    dst = split(jnp.concatenate(
        [edge_index[1], jnp.full((ep - e,), n, jnp.int32)]),
        jnp.int32(n))
    zeros = jnp.zeros((acc_rows, h), jnp.float32)

    segsum = _make_segsum(n, h, k0, k1, acc_rows)

    def gin_dense(h_in, parts, p, relu):
        t, stats = _mlp1(h_in, parts[0, :n], parts[1, :n], p["W1"], p["b1"], n)
        return _mlp2(t, stats, p["g"], p["be"], p["W2"], p["b2"], relu, n)

    p0 = segsum(x, src, dst, zeros)
    h0 = gin_dense(x, p0, params["c0"], True)
    p1 = segsum(h0, src, dst, zeros)
    h1 = gin_dense(h0, p1, params["c1"], True)
    p2 = segsum(h1, src, dst, zeros)
    mean = gin_dense(h1, p2, params["c2"], False)
    p3 = params["c3"]
    t3, st3 = _mlp1(h1, p2[0, :n], p2[1, :n], p3["W1"], p3["b1"], n)
    z = _mlp2(t3, st3, p3["g"], p3["be"], p3["W2"], p3["b2"], False, n,
              mean=mean, noise=gaussian_noise)
    return z


# speedup vs baseline: 1.8856x; 1.8856x over previous
import functools

import jax
import jax.numpy as jnp
from jax import lax
from jax.experimental import pallas as pl
from jax.experimental.pallas import tpu as pltpu
from jax.experimental.pallas import tpu_sc as plsc

NC = 2
NS = 16
NW = NC * NS
CHUNK = 120
NBUF = 3
F0_NUM, F0_DEN = 222, 326

ROW_BLK = 2000


def _make_segsum(n_rows, h, k0, k1, acc_rows):
    mesh = plsc.VectorSubcoreMesh(core_axis_name="c", subcore_axis_name="s")
    rpt = acc_rows // NS

    @functools.partial(
        pl.kernel,
        mesh=mesh,
        out_type=jax.ShapeDtypeStruct((NC, acc_rows, h), jnp.float32),
        scratch_types=[
            pltpu.VMEM((8, 128), jnp.int32),
            pltpu.VMEM((8, 128), jnp.int32),
            pltpu.VMEM((NBUF, CHUNK, h), jnp.float32),
            pltpu.VMEM_SHARED((acc_rows, h), jnp.float32),
        ] + [pltpu.SemaphoreType.DMA] * (4 * NBUF + 1),
    )
    def segsum(table_hbm, src_hbm, dst_hbm, zeros_hbm, out_hbm,
               sring, dring, rows_v, acc, *sems):
        c = lax.axis_index("c")
        s = lax.axis_index("s")
        wid = s * NC + c
        kc = jnp.where(c == 0, k0, k1)
        gsem = sems[:NBUF]
        ssem = sems[NBUF:2 * NBUF]
        dsem = sems[2 * NBUF:3 * NBUF]
        isem = sems[3 * NBUF:4 * NBUF]
        zsem = sems[4 * NBUF]
        zcp = pltpu.make_async_copy(zeros_hbm.at[pl.ds(s * rpt, rpt)],
                                    acc.at[pl.ds(s * rpt, rpt)], zsem)
        zcp.start()

        def sidx_cp(jj, b):
            return pltpu.make_async_copy(src_hbm.at[wid, jj], sring.at[b],
                                         isem[b])

        def didx_cp(jj, b):
            return pltpu.make_async_copy(dst_hbm.at[wid, jj], dring.at[b],
                                         dsem[b])

        def gather_cp(jj, b):
            return pltpu.make_async_copy(
                table_hbm.at[sring.at[b, pl.ds(0, CHUNK)]], rows_v.at[b],
                gsem[b])

        def scatter_cp(b):
            return pltpu.make_async_copy(
                rows_v.at[b], acc.at[dring.at[b, pl.ds(0, CHUNK)]], ssem[b])

        sidx_cp(0, 0).start()
        sidx_cp(1, 1).start()
        sidx_cp(2, 2).start()
        didx_cp(0, 0).start()
        didx_cp(1, 1).start()
        sidx_cp(0, 0).wait()
        gather_cp(0, 0).start()
        sidx_cp(1, 1).wait()
        gather_cp(1, 1).start()
        zcp.wait()
        plsc.subcore_barrier()

        def body(i, carry):
            j = i * NBUF
            for b in range(NBUF):
                jj = j + b
                bn = (b + 2) % NBUF

                @pl.when(jj >= 1)
                def _():
                    scatter_cp(bn).wait()

                gather_cp(jj, b).wait()
                didx_cp(jj, b).wait()
                pltpu.async_copy(
                    rows_v.at[b], acc.at[dring.at[b, pl.ds(0, CHUNK)]],
                    ssem[b], add=True)

                @pl.when(jj + 3 < kc)
                def _():
                    sidx_cp(jj + 3, b).start()

                @pl.when(jj + 2 < kc)
                def _():
                    didx_cp(jj + 2, bn).start()
                    sidx_cp(jj + 2, bn).wait()
                    gather_cp(jj + 2, bn).start()
            return carry

        lax.fori_loop(0, kc // NBUF, body, 0)
        scatter_cp(2).wait()
        plsc.subcore_barrier()
        pltpu.sync_copy(acc.at[pl.ds(s * rpt, rpt)],
                        out_hbm.at[c, pl.ds(s * rpt, rpt)])

    return segsum


def _mlp1_body(x_ref, a0_ref, a1_ref, w_ref, b_ref, t_ref, stats_ref):
    hcols = t_ref.shape[1]
    hid = x_ref[...] + a0_ref[...] + a1_ref[...]
    t = jnp.dot(hid, w_ref[...], preferred_element_type=jnp.float32) + b_ref[...]
    t_ref[...] = t

    @pl.when(pl.program_id(0) == 0)
    def _():
        stats_ref[...] = jnp.zeros_like(stats_ref)

    sums = jnp.concatenate(
        [jnp.sum(t, axis=0, keepdims=True),
         jnp.sum(t * t, axis=0, keepdims=True),
         jnp.zeros((6, hcols), jnp.float32)],
        axis=0,
    )
    stats_ref[...] += sums


def _mlp1(x, a0, a1, w1, b1, n_rows):
    h = x.shape[1]
    h2 = w1.shape[1]
    grid = n_rows // ROW_BLK
    return pl.pallas_call(
        _mlp1_body,
        grid=(grid,),
        in_specs=[
            pl.BlockSpec((ROW_BLK, h), lambda i: (i, 0)),
            pl.BlockSpec((ROW_BLK, h), lambda i: (i, 0)),
            pl.BlockSpec((ROW_BLK, h), lambda i: (i, 0)),
            pl.BlockSpec((h, h2), lambda i: (0, 0)),
            pl.BlockSpec((1, h2), lambda i: (0, 0)),
        ],
        out_specs=[
            pl.BlockSpec((ROW_BLK, h2), lambda i: (i, 0)),
            pl.BlockSpec((8, h2), lambda i: (0, 0)),
        ],
        out_shape=[
            jax.ShapeDtypeStruct((n_rows, h2), jnp.float32),
            jax.ShapeDtypeStruct((8, h2), jnp.float32),
        ],
    )(x, a0, a1, w1, b1.reshape(1, h2))


def _mlp2_body(t_ref, stats_ref, g_ref, be_ref, w_ref, b_ref, o_ref,
               *, relu, n_rows, final):
    inv_n = 1.0 / n_rows
    m = stats_ref[0:1, :] * inv_n
    v = stats_ref[1:2, :] * inv_n - m * m
    scale = lax.rsqrt(v + 1e-5) * g_ref[...]
    hid = (t_ref[...] - m) * scale + be_ref[...]
    if relu:
        hid = jnp.maximum(hid, 0.0)
    o = jnp.dot(hid, w_ref[...], preferred_element_type=jnp.float32) + b_ref[...]
    o_ref[...] = o


def _mlp2_final_body(t_ref, stats_ref, g_ref, be_ref, w_ref, b_ref,
                     mean_ref, noise_ref, o_ref, *, n_rows):
    inv_n = 1.0 / n_rows
    m = stats_ref[0:1, :] * inv_n
    v = stats_ref[1:2, :] * inv_n - m * m
    scale = lax.rsqrt(v + 1e-5) * g_ref[...]
    hid = (t_ref[...] - m) * scale + be_ref[...]
    o = jnp.dot(hid, w_ref[...], preferred_element_type=jnp.float32) + b_ref[...]
    o_ref[...] = noise_ref[...] * jnp.exp(o) + mean_ref[...]


def _mlp2(t, stats, g, be, w2, b2, relu, n_rows, mean=None, noise=None):
    h2 = t.shape[1]
    h = w2.shape[1]
    grid = n_rows // ROW_BLK
    in_specs = [
        pl.BlockSpec((ROW_BLK, h2), lambda i: (i, 0)),
        pl.BlockSpec((8, h2), lambda i: (0, 0)),
        pl.BlockSpec((1, h2), lambda i: (0, 0)),
        pl.BlockSpec((1, h2), lambda i: (0, 0)),
        pl.BlockSpec((h2, h), lambda i: (0, 0)),
        pl.BlockSpec((1, h), lambda i: (0, 0)),
    ]
    args = [t, stats, g.reshape(1, h2), be.reshape(1, h2), w2, b2.reshape(1, h)]
    if mean is None:
        body = functools.partial(_mlp2_body, relu=relu, n_rows=n_rows, final=False)
    else:
        body = functools.partial(_mlp2_final_body, n_rows=n_rows)
        in_specs += [
            pl.BlockSpec((ROW_BLK, h), lambda i: (i, 0)),
            pl.BlockSpec((ROW_BLK, h), lambda i: (i, 0)),
        ]
        args += [mean, noise]
    return pl.pallas_call(
        body,
        grid=(grid,),
        in_specs=in_specs,
        out_specs=pl.BlockSpec((ROW_BLK, h), lambda i: (i, 0)),
        out_shape=jax.ShapeDtypeStruct((n_rows, h), jnp.float32),
    )(*args)


def kernel(x, edge_index, gaussian_noise, params):
    n, h = x.shape
    e = edge_index.shape[1]
    acc_rows = ((n + NS * 8) // (NS * 8)) * (NS * 8)

    epp = -(-e // NS)
    e0 = (epp * F0_NUM) // F0_DEN
    k0 = -(-(-(-e0 // CHUNK)) // NBUF) * NBUF
    e1 = epp - e0
    k1 = -(-(-(-e1 // CHUNK)) // NBUF) * NBUF

    def slab(vals, fill, e_pt, k):
        v = vals.reshape(NS, e_pt)
        v = jnp.pad(v, ((0, 0), (0, k * CHUNK - e_pt)), constant_values=fill)
        v = v.reshape(NS, k, CHUNK)
        v = jnp.pad(v, ((0, 0), (0, k0 - k), (0, 128 - CHUNK)),
                    constant_values=fill)
        return v

    def split(vals, fill):
        a = slab(vals[:NS * e0], fill, e0, k0)
        b = slab(vals[NS * e0:NS * epp], fill, e1, k1)
        return jnp.stack([a, b], axis=1).reshape(NW, k0, 128)

    ep = NS * epp
    src = split(jnp.concatenate(
        [edge_index[0], jnp.zeros((ep - e,), jnp.int32)]), 0)
    dst = split(jnp.concatenate(
        [edge_index[1], jnp.full((ep - e,), n, jnp.int32)]),
        jnp.int32(n))
    zeros = jnp.zeros((acc_rows, h), jnp.float32)

    segsum = _make_segsum(n, h, k0, k1, acc_rows)

    def gin_dense(h_in, parts, p, relu):
        t, stats = _mlp1(h_in, parts[0, :n], parts[1, :n], p["W1"], p["b1"], n)
        return _mlp2(t, stats, p["g"], p["be"], p["W2"], p["b2"], relu, n)

    p0 = segsum(x, src, dst, zeros)
    h0 = gin_dense(x, p0, params["c0"], True)
    p1 = segsum(h0, src, dst, zeros)
    h1 = gin_dense(h0, p1, params["c1"], True)
    p2 = segsum(h1, src, dst, zeros)
    mean = gin_dense(h1, p2, params["c2"], False)
    p3 = params["c3"]
    t3, st3 = _mlp1(h1, p2[0, :n], p2[1, :n], p3["W1"], p3["b1"], n)
    z = _mlp2(t3, st3, p3["g"], p3["be"], p3["W2"], p3["b2"], False, n,
              mean=mean, noise=gaussian_noise)
    return z


# speedup vs baseline: 1.9208x; 1.0187x over previous
import functools

import jax
import jax.numpy as jnp
from jax import lax
from jax.experimental import pallas as pl
from jax.experimental.pallas import tpu as pltpu
from jax.experimental.pallas import tpu_sc as plsc

NC = 2
NS = 16
NW = NC * NS
CHUNK = 120
NBUF = 3
F0_NUM, F0_DEN = 222, 326

ROW_BLK = 2000


def _make_segsum(n_rows, h, k0, k1, acc_rows):
    mesh = plsc.VectorSubcoreMesh(core_axis_name="c", subcore_axis_name="s")
    rpt = acc_rows // NS

    @functools.partial(
        pl.kernel,
        mesh=mesh,
        out_type=jax.ShapeDtypeStruct((NC, acc_rows, h), jnp.float32),
        scratch_types=[
            pltpu.VMEM((8, 128), jnp.int32),
            pltpu.VMEM((8, 128), jnp.int32),
            pltpu.VMEM((NBUF, CHUNK, h), jnp.float32),
            pltpu.VMEM_SHARED((acc_rows, h), jnp.float32),
        ] + [pltpu.SemaphoreType.DMA] * (4 * NBUF + 1),
    )
    def segsum(table_hbm, src_hbm, dst_hbm, zeros_hbm, out_hbm,
               sring, dring, rows_v, acc, *sems):
        c = lax.axis_index("c")
        s = lax.axis_index("s")
        wid = s * NC + c
        kc = jnp.where(c == 0, k0, k1)
        gsem = sems[:NBUF]
        ssem = sems[NBUF:2 * NBUF]
        dsem = sems[2 * NBUF:3 * NBUF]
        isem = sems[3 * NBUF:4 * NBUF]
        zsem = sems[4 * NBUF]
        zcp = pltpu.make_async_copy(zeros_hbm.at[pl.ds(s * rpt, rpt)],
                                    acc.at[pl.ds(s * rpt, rpt)], zsem)
        zcp.start()

        def sidx_cp(jj, b):
            return pltpu.make_async_copy(src_hbm.at[wid, jj], sring.at[b],
                                         isem[b])

        def didx_cp(jj, b):
            return pltpu.make_async_copy(dst_hbm.at[wid, jj], dring.at[b],
                                         dsem[b])

        def gather_cp(jj, b):
            return pltpu.make_async_copy(
                table_hbm.at[sring.at[b, pl.ds(0, CHUNK)]], rows_v.at[b],
                gsem[b])

        def scatter_cp(b):
            return pltpu.make_async_copy(
                rows_v.at[b], acc.at[dring.at[b, pl.ds(0, CHUNK)]], ssem[b])

        sidx_cp(0, 0).start()
        sidx_cp(1, 1).start()
        sidx_cp(2, 2).start()
        didx_cp(0, 0).start()
        didx_cp(1, 1).start()
        sidx_cp(0, 0).wait()
        gather_cp(0, 0).start()
        sidx_cp(1, 1).wait()
        gather_cp(1, 1).start()
        zcp.wait()
        plsc.subcore_barrier()

        def body(i, carry):
            j = i * NBUF
            for b in range(NBUF):
                jj = j + b
                bn = (b + 2) % NBUF

                @pl.when(jj >= 1)
                def _():
                    scatter_cp(bn).wait()

                gather_cp(jj, b).wait()
                didx_cp(jj, b).wait()
                pltpu.async_copy(
                    rows_v.at[b], acc.at[dring.at[b, pl.ds(0, CHUNK)]],
                    ssem[b], add=True)

                @pl.when(jj + 3 < kc)
                def _():
                    sidx_cp(jj + 3, b).start()

                @pl.when(jj + 2 < kc)
                def _():
                    didx_cp(jj + 2, bn).start()
                    sidx_cp(jj + 2, bn).wait()
                    gather_cp(jj + 2, bn).start()
            return carry

        lax.fori_loop(0, kc // NBUF, body, 0)
        scatter_cp(2).wait()
        plsc.subcore_barrier()
        pltpu.sync_copy(acc.at[pl.ds(s * rpt, rpt)],
                        out_hbm.at[c, pl.ds(s * rpt, rpt)])

    return segsum


def _mlp1_body(x_ref, a0_ref, a1_ref, w_ref, b_ref, t_ref, stats_ref):
    hcols = t_ref.shape[1]
    hid = x_ref[...] + a0_ref[...] + a1_ref[...]
    t = jnp.dot(hid, w_ref[...], preferred_element_type=jnp.float32) + b_ref[...]
    t_ref[...] = t

    @pl.when(pl.program_id(0) == 0)
    def _():
        stats_ref[...] = jnp.zeros_like(stats_ref)

    sums = jnp.concatenate(
        [jnp.sum(t, axis=0, keepdims=True),
         jnp.sum(t * t, axis=0, keepdims=True),
         jnp.zeros((6, hcols), jnp.float32)],
        axis=0,
    )
    stats_ref[...] += sums


def _mlp1(x, a0, a1, w1, b1, n_rows):
    h = x.shape[1]
    h2 = w1.shape[1]
    grid = n_rows // ROW_BLK
    return pl.pallas_call(
        _mlp1_body,
        grid=(grid,),
        in_specs=[
            pl.BlockSpec((ROW_BLK, h), lambda i: (i, 0)),
            pl.BlockSpec((ROW_BLK, h), lambda i: (i, 0)),
            pl.BlockSpec((ROW_BLK, h), lambda i: (i, 0)),
            pl.BlockSpec((h, h2), lambda i: (0, 0)),
            pl.BlockSpec((1, h2), lambda i: (0, 0)),
        ],
        out_specs=[
            pl.BlockSpec((ROW_BLK, h2), lambda i: (i, 0)),
            pl.BlockSpec((8, h2), lambda i: (0, 0)),
        ],
        out_shape=[
            jax.ShapeDtypeStruct((n_rows, h2), jnp.float32),
            jax.ShapeDtypeStruct((8, h2), jnp.float32),
        ],
    )(x, a0, a1, w1, b1.reshape(1, h2))


def _mlp2_body(t_ref, stats_ref, g_ref, be_ref, w_ref, b_ref, o_ref,
               *, relu, n_rows, final):
    inv_n = 1.0 / n_rows
    m = stats_ref[0:1, :] * inv_n
    v = stats_ref[1:2, :] * inv_n - m * m
    scale = lax.rsqrt(v + 1e-5) * g_ref[...]
    hid = (t_ref[...] - m) * scale + be_ref[...]
    if relu:
        hid = jnp.maximum(hid, 0.0)
    o = jnp.dot(hid, w_ref[...], preferred_element_type=jnp.float32) + b_ref[...]
    o_ref[...] = o


def _mlp2_final_body(t_ref, stats_ref, g_ref, be_ref, w_ref, b_ref,
                     mean_ref, noise_ref, o_ref, *, n_rows):
    inv_n = 1.0 / n_rows
    m = stats_ref[0:1, :] * inv_n
    v = stats_ref[1:2, :] * inv_n - m * m
    scale = lax.rsqrt(v + 1e-5) * g_ref[...]
    hid = (t_ref[...] - m) * scale + be_ref[...]
    o = jnp.dot(hid, w_ref[...], preferred_element_type=jnp.float32) + b_ref[...]
    o_ref[...] = noise_ref[...] * jnp.exp(o) + mean_ref[...]


def _mlp1_dual_body(x_ref, a0_ref, a1_ref, wm_ref, bm_ref, wl_ref, bl_ref,
                    tm_ref, tl_ref, stats_ref):
    hcols = tm_ref.shape[1]
    hid = x_ref[...] + a0_ref[...] + a1_ref[...]
    tm = jnp.dot(hid, wm_ref[...], preferred_element_type=jnp.float32) + bm_ref[...]
    tl = jnp.dot(hid, wl_ref[...], preferred_element_type=jnp.float32) + bl_ref[...]
    tm_ref[...] = tm
    tl_ref[...] = tl

    @pl.when(pl.program_id(0) == 0)
    def _():
        stats_ref[...] = jnp.zeros_like(stats_ref)

    sums = jnp.concatenate(
        [jnp.sum(tm, axis=0, keepdims=True),
         jnp.sum(tm * tm, axis=0, keepdims=True),
         jnp.sum(tl, axis=0, keepdims=True),
         jnp.sum(tl * tl, axis=0, keepdims=True),
         jnp.zeros((4, hcols), jnp.float32)],
        axis=0,
    )
    stats_ref[...] += sums


def _mlp1_dual(x, a0, a1, wm, bm, wl, bl, n_rows):
    h = x.shape[1]
    h2 = wm.shape[1]
    grid = n_rows // ROW_BLK
    row = pl.BlockSpec((ROW_BLK, h), lambda i: (i, 0))
    mat = pl.BlockSpec((h, h2), lambda i: (0, 0))
    vec = pl.BlockSpec((1, h2), lambda i: (0, 0))
    return pl.pallas_call(
        _mlp1_dual_body,
        grid=(grid,),
        in_specs=[row, row, row, mat, vec, mat, vec],
        out_specs=[
            pl.BlockSpec((ROW_BLK, h2), lambda i: (i, 0)),
            pl.BlockSpec((ROW_BLK, h2), lambda i: (i, 0)),
            pl.BlockSpec((8, h2), lambda i: (0, 0)),
        ],
        out_shape=[
            jax.ShapeDtypeStruct((n_rows, h2), jnp.float32),
            jax.ShapeDtypeStruct((n_rows, h2), jnp.float32),
            jax.ShapeDtypeStruct((8, h2), jnp.float32),
        ],
    )(x, a0, a1, wm, bm.reshape(1, h2), wl, bl.reshape(1, h2))


def _mlp2_dual_final_body(tm_ref, tl_ref, stats_ref, gm_ref, bem_ref, wm_ref,
                          bm2_ref, gl_ref, bel_ref, wl_ref, bl2_ref,
                          noise_ref, o_ref, *, n_rows):
    inv_n = 1.0 / n_rows
    mm = stats_ref[0:1, :] * inv_n
    vm = stats_ref[1:2, :] * inv_n - mm * mm
    ml = stats_ref[2:3, :] * inv_n
    vl = stats_ref[3:4, :] * inv_n - ml * ml
    hm = (tm_ref[...] - mm) * (lax.rsqrt(vm + 1e-5) * gm_ref[...]) + bem_ref[...]
    hl = (tl_ref[...] - ml) * (lax.rsqrt(vl + 1e-5) * gl_ref[...]) + bel_ref[...]
    mean = jnp.dot(hm, wm_ref[...], preferred_element_type=jnp.float32) + bm2_ref[...]
    o = jnp.dot(hl, wl_ref[...], preferred_element_type=jnp.float32) + bl2_ref[...]
    o_ref[...] = noise_ref[...] * jnp.exp(o) + mean


def _mlp2_dual_final(tm, tl, stats, pm, plog, noise, n_rows):
    h2 = tm.shape[1]
    h = pm["W2"].shape[1]
    grid = n_rows // ROW_BLK
    rowt = pl.BlockSpec((ROW_BLK, h2), lambda i: (i, 0))
    mat = pl.BlockSpec((h2, h), lambda i: (0, 0))
    vec2 = pl.BlockSpec((1, h2), lambda i: (0, 0))
    vech = pl.BlockSpec((1, h), lambda i: (0, 0))
    return pl.pallas_call(
        functools.partial(_mlp2_dual_final_body, n_rows=n_rows),
        grid=(grid,),
        in_specs=[rowt, rowt, pl.BlockSpec((8, h2), lambda i: (0, 0)),
                  vec2, vec2, mat, vech, vec2, vec2, mat, vech,
                  pl.BlockSpec((ROW_BLK, h), lambda i: (i, 0))],
        out_specs=pl.BlockSpec((ROW_BLK, h), lambda i: (i, 0)),
        out_shape=jax.ShapeDtypeStruct((n_rows, h), jnp.float32),
    )(tm, tl, stats,
      pm["g"].reshape(1, h2), pm["be"].reshape(1, h2), pm["W2"],
      pm["b2"].reshape(1, h),
      plog["g"].reshape(1, h2), plog["be"].reshape(1, h2), plog["W2"],
      plog["b2"].reshape(1, h),
      noise)


def _mlp2(t, stats, g, be, w2, b2, relu, n_rows, mean=None, noise=None):
    h2 = t.shape[1]
    h = w2.shape[1]
    grid = n_rows // ROW_BLK
    in_specs = [
        pl.BlockSpec((ROW_BLK, h2), lambda i: (i, 0)),
        pl.BlockSpec((8, h2), lambda i: (0, 0)),
        pl.BlockSpec((1, h2), lambda i: (0, 0)),
        pl.BlockSpec((1, h2), lambda i: (0, 0)),
        pl.BlockSpec((h2, h), lambda i: (0, 0)),
        pl.BlockSpec((1, h), lambda i: (0, 0)),
    ]
    args = [t, stats, g.reshape(1, h2), be.reshape(1, h2), w2, b2.reshape(1, h)]
    if mean is None:
        body = functools.partial(_mlp2_body, relu=relu, n_rows=n_rows, final=False)
    else:
        body = functools.partial(_mlp2_final_body, n_rows=n_rows)
        in_specs += [
            pl.BlockSpec((ROW_BLK, h), lambda i: (i, 0)),
            pl.BlockSpec((ROW_BLK, h), lambda i: (i, 0)),
        ]
        args += [mean, noise]
    return pl.pallas_call(
        body,
        grid=(grid,),
        in_specs=in_specs,
        out_specs=pl.BlockSpec((ROW_BLK, h), lambda i: (i, 0)),
        out_shape=jax.ShapeDtypeStruct((n_rows, h), jnp.float32),
    )(*args)


def kernel(x, edge_index, gaussian_noise, params):
    n, h = x.shape
    e = edge_index.shape[1]
    acc_rows = ((n + NS * 8) // (NS * 8)) * (NS * 8)

    epp = -(-e // NS)
    e0 = (epp * F0_NUM) // F0_DEN
    k0 = -(-(-(-e0 // CHUNK)) // NBUF) * NBUF
    e1 = epp - e0
    k1 = -(-(-(-e1 // CHUNK)) // NBUF) * NBUF

    def slab(vals, fill, e_pt, k):
        v = vals.reshape(NS, e_pt)
        v = jnp.pad(v, ((0, 0), (0, k * CHUNK - e_pt)), constant_values=fill)
        v = v.reshape(NS, k, CHUNK)
        v = jnp.pad(v, ((0, 0), (0, k0 - k), (0, 128 - CHUNK)),
                    constant_values=fill)
        return v

    def split(vals, fill):
        a = slab(vals[:NS * e0], fill, e0, k0)
        b = slab(vals[NS * e0:NS * epp], fill, e1, k1)
        return jnp.stack([a, b], axis=1).reshape(NW, k0, 128)

    ep = NS * epp
    src = split(jnp.concatenate(
        [edge_index[0], jnp.zeros((ep - e,), jnp.int32)]), 0)
    dst = split(jnp.concatenate(
        [edge_index[1], jnp.full((ep - e,), n, jnp.int32)]),
        jnp.int32(n))
    zeros = jnp.zeros((acc_rows, h), jnp.float32)

    segsum = _make_segsum(n, h, k0, k1, acc_rows)

    def gin_dense(h_in, parts, p, relu):
        t, stats = _mlp1(h_in, parts[0, :n], parts[1, :n], p["W1"], p["b1"], n)
        return _mlp2(t, stats, p["g"], p["be"], p["W2"], p["b2"], relu, n)

    p0 = segsum(x, src, dst, zeros)
    h0 = gin_dense(x, p0, params["c0"], True)
    p1 = segsum(h0, src, dst, zeros)
    h1 = gin_dense(h0, p1, params["c1"], True)
    p2 = segsum(h1, src, dst, zeros)
    pm, plog = params["c2"], params["c3"]
    tm, tl, st = _mlp1_dual(h1, p2[0, :n], p2[1, :n], pm["W1"], pm["b1"],
                            plog["W1"], plog["b1"], n)
    z = _mlp2_dual_final(tm, tl, st, pm, plog, gaussian_noise, n)
    return z


# speedup vs baseline: 1.9280x; 1.0037x over previous
import functools

import jax
import jax.numpy as jnp
from jax import lax
from jax.experimental import pallas as pl
from jax.experimental.pallas import tpu as pltpu
from jax.experimental.pallas import tpu_sc as plsc

NC = 2
NS = 16
NW = NC * NS
CHUNK = 120
NBUF = 3
F0_NUM, F0_DEN = 222, 326

ROW_BLK = 2000


def _make_segsum(n_rows, h, k0, k1, acc_rows):
    mesh = plsc.VectorSubcoreMesh(core_axis_name="c", subcore_axis_name="s")
    rpt = acc_rows // NS

    @functools.partial(
        pl.kernel,
        mesh=mesh,
        out_type=jax.ShapeDtypeStruct((NC, acc_rows, h), jnp.float32),
        scratch_types=[
            pltpu.VMEM((8, 128), jnp.int32),
            pltpu.VMEM((8, 128), jnp.int32),
            pltpu.VMEM((NBUF, CHUNK, h), jnp.float32),
            pltpu.VMEM_SHARED((acc_rows, h), jnp.float32),
        ] + [pltpu.SemaphoreType.DMA] * (4 * NBUF + 1),
    )
    def segsum(table_hbm, src_hbm, dst_hbm, zeros_hbm, out_hbm,
               sring, dring, rows_v, acc, *sems):
        c = lax.axis_index("c")
        s = lax.axis_index("s")
        wid = s * NC + c
        kc = jnp.where(c == 0, k0, k1)
        gsem = sems[:NBUF]
        ssem = sems[NBUF:2 * NBUF]
        dsem = sems[2 * NBUF:3 * NBUF]
        isem = sems[3 * NBUF:4 * NBUF]
        zsem = sems[4 * NBUF]
        zcp = pltpu.make_async_copy(zeros_hbm.at[pl.ds(s * rpt, rpt)],
                                    acc.at[pl.ds(s * rpt, rpt)], zsem)
        zcp.start()

        def sidx_cp(jj, b):
            return pltpu.make_async_copy(src_hbm.at[wid, jj], sring.at[b],
                                         isem[b])

        def didx_cp(jj, b):
            return pltpu.make_async_copy(dst_hbm.at[wid, jj], dring.at[b],
                                         dsem[b])

        def gather_cp(jj, b):
            return pltpu.make_async_copy(
                table_hbm.at[sring.at[b, pl.ds(0, CHUNK)]], rows_v.at[b],
                gsem[b])

        def scatter_cp(b):
            return pltpu.make_async_copy(
                rows_v.at[b], acc.at[dring.at[b, pl.ds(0, CHUNK)]], ssem[b])

        sidx_cp(0, 0).start()
        sidx_cp(1, 1).start()
        sidx_cp(2, 2).start()
        didx_cp(0, 0).start()
        didx_cp(1, 1).start()
        sidx_cp(0, 0).wait()
        gather_cp(0, 0).start()
        sidx_cp(1, 1).wait()
        gather_cp(1, 1).start()
        zcp.wait()
        plsc.subcore_barrier()

        def body(i, carry):
            j = i * NBUF
            for b in range(NBUF):
                jj = j + b
                bn = (b + 2) % NBUF

                @pl.when(jj >= 1)
                def _():
                    scatter_cp(bn).wait()

                gather_cp(jj, b).wait()
                didx_cp(jj, b).wait()
                pltpu.async_copy(
                    rows_v.at[b], acc.at[dring.at[b, pl.ds(0, CHUNK)]],
                    ssem[b], add=True)

                @pl.when(jj + 3 < kc)
                def _():
                    sidx_cp(jj + 3, b).start()

                @pl.when(jj + 2 < kc)
                def _():
                    didx_cp(jj + 2, bn).start()
                    sidx_cp(jj + 2, bn).wait()
                    gather_cp(jj + 2, bn).start()
            return carry

        lax.fori_loop(0, kc // NBUF, body, 0)
        scatter_cp(2).wait()
        plsc.subcore_barrier()
        pltpu.sync_copy(acc.at[pl.ds(s * rpt, rpt)],
                        out_hbm.at[c, pl.ds(s * rpt, rpt)])

    return segsum


def _mlp1_body(x_ref, a0_ref, a1_ref, w_ref, b_ref, t_ref, stats_ref):
    hcols = t_ref.shape[1]
    hid = x_ref[...] + a0_ref[...] + a1_ref[...]
    t = jnp.dot(hid, w_ref[...], preferred_element_type=jnp.float32) + b_ref[...]
    t_ref[...] = t

    @pl.when(pl.program_id(0) == 0)
    def _():
        stats_ref[...] = jnp.zeros_like(stats_ref)

    sums = jnp.concatenate(
        [jnp.sum(t, axis=0, keepdims=True),
         jnp.sum(t * t, axis=0, keepdims=True),
         jnp.zeros((6, hcols), jnp.float32)],
        axis=0,
    )
    stats_ref[...] += sums


def _mlp1(x, a0, a1, w1, b1, n_rows):
    h = x.shape[1]
    h2 = w1.shape[1]
    grid = n_rows // ROW_BLK
    return pl.pallas_call(
        _mlp1_body,
        grid=(grid,),
        in_specs=[
            pl.BlockSpec((ROW_BLK, h), lambda i: (i, 0)),
            pl.BlockSpec((ROW_BLK, h), lambda i: (i, 0)),
            pl.BlockSpec((ROW_BLK, h), lambda i: (i, 0)),
            pl.BlockSpec((h, h2), lambda i: (0, 0)),
            pl.BlockSpec((1, h2), lambda i: (0, 0)),
        ],
        out_specs=[
            pl.BlockSpec((ROW_BLK, h2), lambda i: (i, 0)),
            pl.BlockSpec((8, h2), lambda i: (0, 0)),
        ],
        out_shape=[
            jax.ShapeDtypeStruct((n_rows, h2), jnp.float32),
            jax.ShapeDtypeStruct((8, h2), jnp.float32),
        ],
    )(x, a0, a1, w1, b1.reshape(1, h2))


def _mlp2_body(t_ref, stats_ref, g_ref, be_ref, w_ref, b_ref, o_ref,
               *, relu, n_rows, final):
    inv_n = 1.0 / n_rows
    m = stats_ref[0:1, :] * inv_n
    v = stats_ref[1:2, :] * inv_n - m * m
    scale = lax.rsqrt(v + 1e-5) * g_ref[...]
    hid = (t_ref[...] - m) * scale + be_ref[...]
    if relu:
        hid = jnp.maximum(hid, 0.0)
    o = jnp.dot(hid, w_ref[...], preferred_element_type=jnp.float32) + b_ref[...]
    o_ref[...] = o


def _mlp2_final_body(t_ref, stats_ref, g_ref, be_ref, w_ref, b_ref,
                     mean_ref, noise_ref, o_ref, *, n_rows):
    inv_n = 1.0 / n_rows
    m = stats_ref[0:1, :] * inv_n
    v = stats_ref[1:2, :] * inv_n - m * m
    scale = lax.rsqrt(v + 1e-5) * g_ref[...]
    hid = (t_ref[...] - m) * scale + be_ref[...]
    o = jnp.dot(hid, w_ref[...], preferred_element_type=jnp.float32) + b_ref[...]
    o_ref[...] = noise_ref[...] * jnp.exp(o) + mean_ref[...]


def _mlp1_dual_body(x_ref, a0_ref, a1_ref, wm_ref, bm_ref, wl_ref, bl_ref,
                    tm_ref, tl_ref, stats_ref):
    hcols = tm_ref.shape[1]
    hid = x_ref[...] + a0_ref[...] + a1_ref[...]
    tm = jnp.dot(hid, wm_ref[...], preferred_element_type=jnp.float32) + bm_ref[...]
    tl = jnp.dot(hid, wl_ref[...], preferred_element_type=jnp.float32) + bl_ref[...]
    tm_ref[...] = tm
    tl_ref[...] = tl

    @pl.when(pl.program_id(0) == 0)
    def _():
        stats_ref[...] = jnp.zeros_like(stats_ref)

    sums = jnp.concatenate(
        [jnp.sum(tm, axis=0, keepdims=True),
         jnp.sum(tm * tm, axis=0, keepdims=True),
         jnp.sum(tl, axis=0, keepdims=True),
         jnp.sum(tl * tl, axis=0, keepdims=True),
         jnp.zeros((4, hcols), jnp.float32)],
        axis=0,
    )
    stats_ref[...] += sums


def _mlp1_dual(x, a0, a1, wm, bm, wl, bl, n_rows):
    h = x.shape[1]
    h2 = wm.shape[1]
    grid = n_rows // ROW_BLK
    row = pl.BlockSpec((ROW_BLK, h), lambda i: (i, 0))
    mat = pl.BlockSpec((h, h2), lambda i: (0, 0))
    vec = pl.BlockSpec((1, h2), lambda i: (0, 0))
    return pl.pallas_call(
        _mlp1_dual_body,
        grid=(grid,),
        in_specs=[row, row, row, mat, vec, mat, vec],
        out_specs=[
            pl.BlockSpec((ROW_BLK, h2), lambda i: (i, 0)),
            pl.BlockSpec((ROW_BLK, h2), lambda i: (i, 0)),
            pl.BlockSpec((8, h2), lambda i: (0, 0)),
        ],
        out_shape=[
            jax.ShapeDtypeStruct((n_rows, h2), jnp.float32),
            jax.ShapeDtypeStruct((n_rows, h2), jnp.float32),
            jax.ShapeDtypeStruct((8, h2), jnp.float32),
        ],
    )(x, a0, a1, wm, bm.reshape(1, h2), wl, bl.reshape(1, h2))


def _mlp2_dual_final_body(tm_ref, tl_ref, stats_ref, gm_ref, bem_ref, wm_ref,
                          bm2_ref, gl_ref, bel_ref, wl_ref, bl2_ref,
                          noise_ref, o_ref, *, n_rows):
    inv_n = 1.0 / n_rows
    mm = stats_ref[0:1, :] * inv_n
    vm = stats_ref[1:2, :] * inv_n - mm * mm
    ml = stats_ref[2:3, :] * inv_n
    vl = stats_ref[3:4, :] * inv_n - ml * ml
    hm = (tm_ref[...] - mm) * (lax.rsqrt(vm + 1e-5) * gm_ref[...]) + bem_ref[...]
    hl = (tl_ref[...] - ml) * (lax.rsqrt(vl + 1e-5) * gl_ref[...]) + bel_ref[...]
    mean = jnp.dot(hm, wm_ref[...], preferred_element_type=jnp.float32) + bm2_ref[...]
    o = jnp.dot(hl, wl_ref[...], preferred_element_type=jnp.float32) + bl2_ref[...]
    o_ref[...] = noise_ref[...] * jnp.exp(o) + mean


def _mlp2_dual_final(tm, tl, stats, pm, plog, noise, n_rows):
    h2 = tm.shape[1]
    h = pm["W2"].shape[1]
    grid = n_rows // ROW_BLK
    rowt = pl.BlockSpec((ROW_BLK, h2), lambda i: (i, 0))
    mat = pl.BlockSpec((h2, h), lambda i: (0, 0))
    vec2 = pl.BlockSpec((1, h2), lambda i: (0, 0))
    vech = pl.BlockSpec((1, h), lambda i: (0, 0))
    return pl.pallas_call(
        functools.partial(_mlp2_dual_final_body, n_rows=n_rows),
        grid=(grid,),
        in_specs=[rowt, rowt, pl.BlockSpec((8, h2), lambda i: (0, 0)),
                  vec2, vec2, mat, vech, vec2, vec2, mat, vech,
                  pl.BlockSpec((ROW_BLK, h), lambda i: (i, 0))],
        out_specs=pl.BlockSpec((ROW_BLK, h), lambda i: (i, 0)),
        out_shape=jax.ShapeDtypeStruct((n_rows, h), jnp.float32),
    )(tm, tl, stats,
      pm["g"].reshape(1, h2), pm["be"].reshape(1, h2), pm["W2"],
      pm["b2"].reshape(1, h),
      plog["g"].reshape(1, h2), plog["be"].reshape(1, h2), plog["W2"],
      plog["b2"].reshape(1, h),
      noise)


def _mlp2(t, stats, g, be, w2, b2, relu, n_rows, mean=None, noise=None):
    h2 = t.shape[1]
    h = w2.shape[1]
    grid = n_rows // ROW_BLK
    in_specs = [
        pl.BlockSpec((ROW_BLK, h2), lambda i: (i, 0)),
        pl.BlockSpec((8, h2), lambda i: (0, 0)),
        pl.BlockSpec((1, h2), lambda i: (0, 0)),
        pl.BlockSpec((1, h2), lambda i: (0, 0)),
        pl.BlockSpec((h2, h), lambda i: (0, 0)),
        pl.BlockSpec((1, h), lambda i: (0, 0)),
    ]
    args = [t, stats, g.reshape(1, h2), be.reshape(1, h2), w2, b2.reshape(1, h)]
    if mean is None:
        body = functools.partial(_mlp2_body, relu=relu, n_rows=n_rows, final=False)
    else:
        body = functools.partial(_mlp2_final_body, n_rows=n_rows)
        in_specs += [
            pl.BlockSpec((ROW_BLK, h), lambda i: (i, 0)),
            pl.BlockSpec((ROW_BLK, h), lambda i: (i, 0)),
        ]
        args += [mean, noise]
    return pl.pallas_call(
        body,
        grid=(grid,),
        in_specs=in_specs,
        out_specs=pl.BlockSpec((ROW_BLK, h), lambda i: (i, 0)),
        out_shape=jax.ShapeDtypeStruct((n_rows, h), jnp.float32),
    )(*args)


def kernel(x, edge_index, gaussian_noise, params):
    n, h = x.shape
    e = edge_index.shape[1]
    acc_rows = ((n + NS * 8) // (NS * 8)) * (NS * 8)

    epp = -(-e // NS)
    e0 = (epp * F0_NUM) // F0_DEN
    k0 = -(-(-(-e0 // CHUNK)) // NBUF) * NBUF
    e1 = epp - e0
    k1 = -(-(-(-e1 // CHUNK)) // NBUF) * NBUF

    def slab(vals, fill, e_pt, k):
        v = vals.reshape(NS, e_pt)
        pad = jnp.broadcast_to(fill[:, None], (NS, k * CHUNK - e_pt))
        v = jnp.concatenate([v, pad], axis=1).reshape(NS, k, CHUNK)
        v = jnp.pad(v, ((0, 0), (0, k0 - k), (0, 128 - CHUNK)))
        return v

    def split(vals, fill):
        a = slab(vals[:NS * e0], fill, e0, k0)
        b = slab(vals[NS * e0:NS * epp], fill, e1, k1)
        return jnp.stack([a, b], axis=1).reshape(NW, k0, 128)

    ep = NS * epp
    src = split(jnp.concatenate(
        [edge_index[0], jnp.zeros((ep - e,), jnp.int32)]),
        jnp.zeros((NS,), jnp.int32))
    dst = split(jnp.concatenate(
        [edge_index[1], jnp.full((ep - e,), n, jnp.int32)]),
        n + jnp.arange(NS, dtype=jnp.int32))
    zeros = jnp.zeros((acc_rows, h), jnp.float32)

    segsum = _make_segsum(n, h, k0, k1, acc_rows)

    def gin_dense(h_in, parts, p, relu):
        t, stats = _mlp1(h_in, parts[0, :n], parts[1, :n], p["W1"], p["b1"], n)
        return _mlp2(t, stats, p["g"], p["be"], p["W2"], p["b2"], relu, n)

    p0 = segsum(x, src, dst, zeros)
    h0 = gin_dense(x, p0, params["c0"], True)
    p1 = segsum(h0, src, dst, zeros)
    h1 = gin_dense(h0, p1, params["c1"], True)
    p2 = segsum(h1, src, dst, zeros)
    pm, plog = params["c2"], params["c3"]
    tm, tl, st = _mlp1_dual(h1, p2[0, :n], p2[1, :n], pm["W1"], pm["b1"],
                            plog["W1"], plog["b1"], n)
    z = _mlp2_dual_final(tm, tl, st, pm, plog, gaussian_noise, n)
    return z
